# baseline (speedup 1.0000x reference)
import functools

import jax
import jax.numpy as jnp
from jax import lax
from jax.experimental import pallas as pl
from jax.experimental.pallas import tpu as pltpu
from jax.experimental.pallas import tpu_sc as plsc

E = 160000
T = 640000
H = 256
I_DIM = 64
NR = 6
SBF_DIM = 42

F32 = jnp.float32
I32 = jnp.int32


def _silu(v):
    return v * jax.nn.sigmoid(v)


_BE = 1600


def _pre_body(x_ref, rbf_ref, wji, bji, wkj, bkj, wr1, wr2, wdown, xji_out, tbl_out):
    xb = x_ref[...]
    x_ji = _silu(jnp.dot(xb, wji[...], preferred_element_type=F32) + bji[...])
    x_kj = _silu(jnp.dot(xb, wkj[...], preferred_element_type=F32) + bkj[...])
    r = jnp.dot(jnp.dot(rbf_ref[...], wr1[...], preferred_element_type=F32),
                wr2[...], preferred_element_type=F32)
    tbl = _silu(jnp.dot(x_kj * r, wdown[...], preferred_element_type=F32))
    xji_out[...] = x_ji
    tbl_out[...] = tbl


def _pre(x, rbf, p):
    full = lambda shape: pl.BlockSpec(shape, lambda i: (0, 0))
    return pl.pallas_call(
        _pre_body,
        grid=(E // _BE,),
        in_specs=[
            pl.BlockSpec((_BE, H), lambda i: (i, 0)),
            pl.BlockSpec((_BE, NR), lambda i: (i, 0)),
            full((H, H)), full((1, H)),
            full((H, H)), full((1, H)),
            full((NR, 8)), full((8, H)),
            full((H, I_DIM)),
        ],
        out_specs=[
            pl.BlockSpec((_BE, H), lambda i: (i, 0)),
            pl.BlockSpec((_BE, I_DIM), lambda i: (i, 0)),
        ],
        out_shape=[
            jax.ShapeDtypeStruct((E, H), F32),
            jax.ShapeDtypeStruct((E, I_DIM), F32),
        ],
    )(x, rbf, p['W_ji'], p['b_ji'].reshape(1, H), p['W_kj'], p['b_kj'].reshape(1, H),
      p['W_rbf1'], p['W_rbf2'], p['W_down'])


_BT = 2000


def _sfeat_body(sbf_ref, w1, w2, out_ref):
    out_ref[...] = jnp.dot(jnp.dot(sbf_ref[...], w1[...], preferred_element_type=F32),
                           w2[...], preferred_element_type=F32)


_T2 = T // 2


def _sfeat(sbf, p, part):
    nblk = _T2 // _BT
    return pl.pallas_call(
        _sfeat_body,
        grid=(nblk,),
        in_specs=[
            pl.BlockSpec((_BT, SBF_DIM), lambda i: (i + part * nblk, 0)),
            pl.BlockSpec((SBF_DIM, 8), lambda i: (0, 0)),
            pl.BlockSpec((8, 128), lambda i: (0, 0)),
        ],
        out_specs=pl.BlockSpec((_BT, 128), lambda i: (i, 0)),
        out_shape=jax.ShapeDtypeStruct((_T2, 128), F32),
    )(sbf, p['W_sbf1'],
      jnp.concatenate([p['W_sbf2'], jnp.zeros((8, 128 - I_DIM), F32)], axis=1))


_NC = 2
_NS = 16
_NW = _NC * _NS
_TPW = T // _NW
_TPW2 = _TPW // 2
_K = 200
_NB = _TPW2 // _K

_CRH = 81920
_TRASH = 512
_CRHP = _CRH + _TRASH
_EP = _NC * _CRH


def _sc_mul_body(part, tbl_hbm, s_hbm, kj_hbm, m_hbm,
                 kjbuf0, kjbuf1, trows0, trows1, srows0, srows1, prod0, prod1,
                 sem_i0, sem_i1, sem_g0, sem_g1, sem_s0, sem_s1, sem_w0, sem_w1):
    cid = lax.axis_index("c")
    sid = lax.axis_index("s")
    wid = cid * _NS + sid
    base = wid * _TPW2
    gbase = part * _T2 + base
    kjb = (kjbuf0, kjbuf1)
    trw = (trows0, trows1)
    srw = (srows0, srows1)
    prd = (prod0, prod1)
    sem_i = (sem_i0, sem_i1)
    sem_g = (sem_g0, sem_g1)
    sem_s = (sem_s0, sem_s1)
    sem_w = (sem_w0, sem_w1)

    def issue_reads(b, par):
        tb = base + b * _K
        pltpu.async_copy(kj_hbm.at[pl.ds(gbase + b * _K, _K)], kjb[par], sem_i[par]).wait()
        pltpu.async_copy(tbl_hbm.at[kjb[par]], trw[par], sem_g[par])
        pltpu.async_copy(s_hbm.at[pl.ds(tb, _K), pl.ds(0, I_DIM)], srw[par], sem_s[par])

    def do_batch(b, par, first, last):
        tb = base + b * _K
        pltpu.make_async_copy(tbl_hbm.at[kjb[par]], trw[par], sem_g[par]).wait()
        pltpu.make_async_copy(s_hbm.at[pl.ds(tb, _K), pl.ds(0, I_DIM)], srw[par],
                              sem_s[par]).wait()

        @pl.when(jnp.logical_not(first))
        def _():
            for q in range(I_DIM // 16):
                pltpu.make_async_copy(prd[par].at[:, pl.ds(q * 16, 16)],
                                      m_hbm.at[q, pl.ds(0, _K)], sem_w[par]).wait()

        trows = trw[par]
        srows = srw[par]
        prod = prd[par]

        def mul_body(r4, carry2):
            rb = r4 * 4
            for dr in range(4):
                for c4 in range(I_DIM // 16):
                    sl = pl.ds(c4 * 16, 16)
                    prod[rb + dr, sl] = trows[rb + dr, sl] * srows[rb + dr, sl]
            return carry2

        lax.fori_loop(0, _K // 4, mul_body, 0)

        for q in range(I_DIM // 16):
            pltpu.async_copy(prod.at[:, pl.ds(q * 16, 16)], m_hbm.at[q, pl.ds(tb, _K)],
                             sem_w[par])

        @pl.when(last)
        def _():
            for q in range(I_DIM // 16):
                pltpu.make_async_copy(prd[par].at[:, pl.ds(q * 16, 16)],
                                      m_hbm.at[q, pl.ds(0, _K)], sem_w[par]).wait()

    issue_reads(0, 0)

    def pair_body(b2, carry):
        b = b2 * 2

        @pl.when(b + 1 < _NB)
        def _():
            issue_reads(b + 1, 1)

        do_batch(b, 0, b == 0, b + 2 >= _NB)

        @pl.when(b + 2 < _NB)
        def _():
            issue_reads(b + 2, 0)

        @pl.when(b + 1 < _NB)
        def _():
            do_batch(b + 1, 1, b + 1 == 1, b + 3 >= _NB)

        return carry

    lax.fori_loop(0, (_NB + 1) // 2, pair_body, 0)


def _sc_mul(tbl, s, idx_kj, part):
    mesh = plsc.VectorSubcoreMesh(core_axis_name="c", subcore_axis_name="s")
    kern = functools.partial(
        pl.kernel,
        mesh=mesh,
        out_type=jax.ShapeDtypeStruct((4, _T2, 16), F32),
        compiler_params=pltpu.CompilerParams(use_tc_tiling_on_sc=False),
        scratch_types=[
            pltpu.VMEM((_K,), I32),
            pltpu.VMEM((_K,), I32),
            pltpu.VMEM((_K, I_DIM), F32),
            pltpu.VMEM((_K, I_DIM), F32),
            pltpu.VMEM((_K, I_DIM), F32),
            pltpu.VMEM((_K, I_DIM), F32),
            pltpu.VMEM((_K, I_DIM), F32),
            pltpu.VMEM((_K, I_DIM), F32),
            pltpu.SemaphoreType.DMA,
            pltpu.SemaphoreType.DMA,
            pltpu.SemaphoreType.DMA,
            pltpu.SemaphoreType.DMA,
            pltpu.SemaphoreType.DMA,
            pltpu.SemaphoreType.DMA,
            pltpu.SemaphoreType.DMA,
            pltpu.SemaphoreType.DMA,
        ],
    )(functools.partial(_sc_mul_body, part))
    return kern(tbl, s, idx_kj)


_ZR = 644
_KA = 400
_NBA = _TPW2 // _KA


def _sc_agg_body(m0_hbm, m1_hbm, ji_hbm, agg_hbm,
                 jb0, jb1, db0, db1, mr0, mr1, zbuf, chunk,
                 sem_m0, sem_m1, sem_j0, sem_j1, sem_w0, sem_w1):
    cid = lax.axis_index("c")
    sid = lax.axis_index("s")
    wid = cid * _NS + sid
    base = wid * _TPW2
    lo = cid * _CRH
    zeros16 = jnp.zeros((16,), F32)
    jb = (jb0, jb1)
    db = (db0, db1)
    mr = (mr0, mr1)
    sem_m = (sem_m0, sem_m1)
    sem_j = (sem_j0, sem_j1)
    sem_w = (sem_w0, sem_w1)

    def zb_body(r2, carry):
        zbuf[r2, pl.ds(0, 16)] = zeros16
        return carry

    lax.fori_loop(0, _ZR, zb_body, 0)

    rows_zero = _CRHP // _NS
    rows_out = _CRH // _NS

    def quarter_body(q, carry0):
        for z in range(rows_zero // _ZR):
            pltpu.sync_copy(zbuf, chunk.at[pl.ds(sid * rows_zero + z * _ZR, _ZR)])
        plsc.subcore_barrier()

        for h, m_hbm in ((0, m0_hbm), (1, m1_hbm)):
            goff = h * _T2

            def issue_reads(b, par, drain):
                tb = base + b * _KA

                @pl.when(drain)
                def _():
                    pltpu.make_async_copy(mr[par], chunk.at[db[par]], sem_w[par]).wait()

                pltpu.async_copy(m_hbm.at[q, pl.ds(tb, _KA)], mr[par], sem_m[par])
                pltpu.async_copy(ji_hbm.at[pl.ds(goff + tb, _KA)], jb[par], sem_j[par])

            def do_batch(b, par):
                tb = base + b * _KA
                pltpu.make_async_copy(m_hbm.at[q, pl.ds(tb, _KA)], mr[par],
                                      sem_m[par]).wait()
                pltpu.make_async_copy(ji_hbm.at[pl.ds(goff + tb, _KA)], jb[par],
                                      sem_j[par]).wait()
                for g in range(_KA // 16):
                    ji = jb[par][pl.ds(g * 16, 16)]
                    rel = ji - lo
                    dst = jnp.where((rel >= 0) & (rel < _CRH), rel,
                                    _CRH + (ji & (_TRASH - 1)))
                    db[par][pl.ds(g * 16, 16)] = dst
                pltpu.async_copy(mr[par], chunk.at[db[par]], sem_w[par], add=True)

            issue_reads(0, 0, jnp.bool_(False))

            def pair_body(b2, carry):
                b = b2 * 2

                @pl.when(b + 1 < _NBA)
                def _():
                    issue_reads(b + 1, 1, b2 > 0)

                do_batch(b, 0)

                @pl.when(b + 2 < _NBA)
                def _():
                    issue_reads(b + 2, 0, jnp.bool_(True))

                @pl.when(b + 1 < _NBA)
                def _():
                    do_batch(b + 1, 1)

                return carry

            lax.fori_loop(0, (_NBA + 1) // 2, pair_body, 0)
            pltpu.make_async_copy(mr[0], chunk.at[db[0]], sem_w[0]).wait()
            pltpu.make_async_copy(mr[1], chunk.at[db[1]], sem_w[1]).wait()
        plsc.subcore_barrier()

        pltpu.sync_copy(chunk.at[pl.ds(sid * rows_out, rows_out)],
                        agg_hbm.at[pl.ds(lo + sid * rows_out, rows_out), pl.ds(q * 16, 16)])
        plsc.subcore_barrier()
        return carry0

    lax.fori_loop(0, I_DIM // 16, quarter_body, 0)


def _sc_agg(m0, m1, idx_ji):
    mesh = plsc.VectorSubcoreMesh(core_axis_name="c", subcore_axis_name="s")
    kern = functools.partial(
        pl.kernel,
        mesh=mesh,
        out_type=jax.ShapeDtypeStruct((_EP, 128), F32),
        compiler_params=pltpu.CompilerParams(use_tc_tiling_on_sc=False),
        scratch_types=[
            pltpu.VMEM((_KA,), I32),
            pltpu.VMEM((_KA,), I32),
            pltpu.VMEM((_KA,), I32),
            pltpu.VMEM((_KA,), I32),
            pltpu.VMEM((_KA, 16), F32),
            pltpu.VMEM((_KA, 16), F32),
            pltpu.VMEM((_ZR, 16), F32),
            pltpu.VMEM_SHARED((_CRHP, 16), F32),
            pltpu.SemaphoreType.DMA,
            pltpu.SemaphoreType.DMA,
            pltpu.SemaphoreType.DMA,
            pltpu.SemaphoreType.DMA,
            pltpu.SemaphoreType.DMA,
            pltpu.SemaphoreType.DMA,
        ],
    )(_sc_agg_body)
    return kern(m0, m1, idx_ji)


def _post_body(agg_ref, xji_ref, x_ref, wup, wbs1, bbs1, wbs2, bbs2,
               wlin, blin, wa1a, ba1a, wa1b, ba1b, wa2a, ba2a, wa2b, ba2b,
               out_ref):
    def mm(a, w, b=None):
        r = jnp.dot(a, w if isinstance(w, jax.Array) else w[...],
                    preferred_element_type=F32)
        if b is not None:
            r = r + b[...]
        return r

    h = xji_ref[...] + _silu(mm(agg_ref[...][:, :I_DIM], wup[...]))
    h = h + _silu(mm(_silu(mm(h, wbs1, bbs1)), wbs2, bbs2))
    h = _silu(mm(h, wlin, blin)) + x_ref[...]
    h = h + _silu(mm(_silu(mm(h, wa1a, ba1a)), wa1b, ba1b))
    h = h + _silu(mm(_silu(mm(h, wa2a, ba2a)), wa2b, ba2b))
    out_ref[...] = h


def _post(agg, xji, x, p):
    full = lambda shape: pl.BlockSpec(shape, lambda i: (0, 0))
    wspecs = []
    wvals = []
    for wn, bn in (('W_bs1', 'b_bs1'), ('W_bs2', 'b_bs2'), ('W_lin', 'b_lin'),
                   ('W_as1a', 'b_as1a'), ('W_as1b', 'b_as1b'),
                   ('W_as2a', 'b_as2a'), ('W_as2b', 'b_as2b')):
        wspecs += [full((H, H)), full((1, H))]
        wvals += [p[wn], p[bn].reshape(1, H)]
    return pl.pallas_call(
        _post_body,
        grid=(E // _BE,),
        in_specs=[
            pl.BlockSpec((_BE, 128), lambda i: (i, 0)),
            pl.BlockSpec((_BE, H), lambda i: (i, 0)),
            pl.BlockSpec((_BE, H), lambda i: (i, 0)),
            full((I_DIM, H)),
        ] + wspecs,
        out_specs=pl.BlockSpec((_BE, H), lambda i: (i, 0)),
        out_shape=jax.ShapeDtypeStruct((E, H), F32),
    )(agg, xji, x, p['W_up'], *wvals)


def kernel(x, rbf, sbf, idx_kj, idx_ji, params):
    xji, tbl = _pre(x, rbf, params)
    kj = idx_kj.astype(I32)
    s0 = _sfeat(sbf, params, 0)
    m0 = _sc_mul(tbl, s0, kj, 0)
    s1 = _sfeat(sbf, params, 1)
    m1 = _sc_mul(tbl, s1, kj, 1)
    agg = _sc_agg(m0, m1, idx_ji.astype(I32))
    return _post(agg, xji, x, params)


# speedup vs baseline: 1.1385x; 1.1385x over previous
import functools

import jax
import jax.numpy as jnp
from jax import lax
from jax.experimental import pallas as pl
from jax.experimental.pallas import tpu as pltpu
from jax.experimental.pallas import tpu_sc as plsc

E = 160000
T = 640000
H = 256
I_DIM = 64
NR = 6
SBF_DIM = 42

F32 = jnp.float32
I32 = jnp.int32


def _silu(v):
    return v * jax.nn.sigmoid(v)


_BE = 1600


def _pre_body(x_ref, rbf_ref, wji, bji, wkj, bkj, wr1, wr2, wdown, xji_out, tbl_out):
    xb = x_ref[...]
    x_ji = _silu(jnp.dot(xb, wji[...], preferred_element_type=F32) + bji[...])
    x_kj = _silu(jnp.dot(xb, wkj[...], preferred_element_type=F32) + bkj[...])
    r = jnp.dot(jnp.dot(rbf_ref[...], wr1[...], preferred_element_type=F32),
                wr2[...], preferred_element_type=F32)
    tbl = _silu(jnp.dot(x_kj * r, wdown[...], preferred_element_type=F32))
    xji_out[...] = x_ji
    tbl_out[...] = tbl


def _pre(x, rbf, p):
    full = lambda shape: pl.BlockSpec(shape, lambda i: (0, 0))
    return pl.pallas_call(
        _pre_body,
        grid=(E // _BE,),
        in_specs=[
            pl.BlockSpec((_BE, H), lambda i: (i, 0)),
            pl.BlockSpec((_BE, NR), lambda i: (i, 0)),
            full((H, H)), full((1, H)),
            full((H, H)), full((1, H)),
            full((NR, 8)), full((8, H)),
            full((H, I_DIM)),
        ],
        out_specs=[
            pl.BlockSpec((_BE, H), lambda i: (i, 0)),
            pl.BlockSpec((_BE, I_DIM), lambda i: (i, 0)),
        ],
        out_shape=[
            jax.ShapeDtypeStruct((E, H), F32),
            jax.ShapeDtypeStruct((E, I_DIM), F32),
        ],
    )(x, rbf, p['W_ji'], p['b_ji'].reshape(1, H), p['W_kj'], p['b_kj'].reshape(1, H),
      p['W_rbf1'], p['W_rbf2'], p['W_down'])


_BT = 2000


def _sfeat_body(sbf_ref, w1, w2, out_ref):
    out_ref[...] = jnp.dot(jnp.dot(sbf_ref[...], w1[...], preferred_element_type=F32),
                           w2[...], preferred_element_type=F32)


_T2 = T // 2


def _sfeat(sbf, p, part):
    nblk = _T2 // _BT
    return pl.pallas_call(
        _sfeat_body,
        grid=(nblk,),
        in_specs=[
            pl.BlockSpec((_BT, SBF_DIM), lambda i: (i + part * nblk, 0)),
            pl.BlockSpec((SBF_DIM, 8), lambda i: (0, 0)),
            pl.BlockSpec((8, 128), lambda i: (0, 0)),
        ],
        out_specs=pl.BlockSpec((_BT, 128), lambda i: (i, 0)),
        out_shape=jax.ShapeDtypeStruct((_T2, 128), F32),
    )(sbf, p['W_sbf1'],
      jnp.concatenate([p['W_sbf2'], jnp.zeros((8, 128 - I_DIM), F32)], axis=1))


_NC = 2
_NS = 16
_NW = _NC * _NS
_TPW = T // _NW
_TPW2 = _TPW // 2
_K = 200
_NB = _TPW2 // _K

_CRH = 81920
_TRASH = 512
_CRHP = _CRH + _TRASH
_EP = _NC * _CRH


def _sc_mul_body(part, tbl_hbm, s_hbm, kj_hbm, m_hbm,
                 kjbuf0, kjbuf1, trows0, trows1, srows0, srows1, prod0, prod1,
                 sem_i0, sem_i1, sem_g0, sem_g1, sem_s0, sem_s1, sem_w0, sem_w1):
    cid = lax.axis_index("c")
    sid = lax.axis_index("s")
    wid = cid * _NS + sid
    base = wid * _TPW2
    gbase = part * _T2 + base
    kjb = (kjbuf0, kjbuf1)
    trw = (trows0, trows1)
    srw = (srows0, srows1)
    prd = (prod0, prod1)
    sem_i = (sem_i0, sem_i1)
    sem_g = (sem_g0, sem_g1)
    sem_s = (sem_s0, sem_s1)
    sem_w = (sem_w0, sem_w1)

    def issue_reads(b, par):
        tb = base + b * _K
        pltpu.async_copy(kj_hbm.at[pl.ds(gbase + b * _K, _K)], kjb[par], sem_i[par]).wait()
        pltpu.async_copy(tbl_hbm.at[kjb[par]], trw[par], sem_g[par])
        pltpu.async_copy(s_hbm.at[pl.ds(tb, _K), pl.ds(0, I_DIM)], srw[par], sem_s[par])

    def do_batch(b, par, first, last):
        tb = base + b * _K
        pltpu.make_async_copy(tbl_hbm.at[kjb[par]], trw[par], sem_g[par]).wait()
        pltpu.make_async_copy(s_hbm.at[pl.ds(tb, _K), pl.ds(0, I_DIM)], srw[par],
                              sem_s[par]).wait()

        @pl.when(jnp.logical_not(first))
        def _():
            pltpu.make_async_copy(prd[par], m_hbm.at[pl.ds(0, _K)], sem_w[par]).wait()

        trows = trw[par]
        srows = srw[par]
        prod = prd[par]

        def mul_body(r4, carry2):
            rb = r4 * 4
            for dr in range(4):
                for c4 in range(I_DIM // 16):
                    sl = pl.ds(c4 * 16, 16)
                    prod[rb + dr, sl] = trows[rb + dr, sl] * srows[rb + dr, sl]
            return carry2

        lax.fori_loop(0, _K // 4, mul_body, 0)

        pltpu.async_copy(prod, m_hbm.at[pl.ds(tb, _K)], sem_w[par])

        @pl.when(last)
        def _():
            pltpu.make_async_copy(prd[par], m_hbm.at[pl.ds(0, _K)], sem_w[par]).wait()

    issue_reads(0, 0)

    def pair_body(b2, carry):
        b = b2 * 2

        @pl.when(b + 1 < _NB)
        def _():
            issue_reads(b + 1, 1)

        do_batch(b, 0, b == 0, b + 2 >= _NB)

        @pl.when(b + 2 < _NB)
        def _():
            issue_reads(b + 2, 0)

        @pl.when(b + 1 < _NB)
        def _():
            do_batch(b + 1, 1, b + 1 == 1, b + 3 >= _NB)

        return carry

    lax.fori_loop(0, (_NB + 1) // 2, pair_body, 0)


def _sc_mul(tbl, s, idx_kj, part):
    mesh = plsc.VectorSubcoreMesh(core_axis_name="c", subcore_axis_name="s")
    kern = functools.partial(
        pl.kernel,
        mesh=mesh,
        out_type=jax.ShapeDtypeStruct((_T2, I_DIM), F32),
        compiler_params=pltpu.CompilerParams(use_tc_tiling_on_sc=False),
        scratch_types=[
            pltpu.VMEM((_K,), I32),
            pltpu.VMEM((_K,), I32),
            pltpu.VMEM((_K, I_DIM), F32),
            pltpu.VMEM((_K, I_DIM), F32),
            pltpu.VMEM((_K, I_DIM), F32),
            pltpu.VMEM((_K, I_DIM), F32),
            pltpu.VMEM((_K, I_DIM), F32),
            pltpu.VMEM((_K, I_DIM), F32),
            pltpu.SemaphoreType.DMA,
            pltpu.SemaphoreType.DMA,
            pltpu.SemaphoreType.DMA,
            pltpu.SemaphoreType.DMA,
            pltpu.SemaphoreType.DMA,
            pltpu.SemaphoreType.DMA,
            pltpu.SemaphoreType.DMA,
            pltpu.SemaphoreType.DMA,
        ],
    )(functools.partial(_sc_mul_body, part))
    return kern(tbl, s, idx_kj)


_ZR = 644
_KA = 400
_NBA = _TPW2 // _KA


def _sc_agg_body(m0_hbm, m1_hbm, ji_hbm, agg_hbm,
                 jb0, jb1, db0, db1, mr0, mr1, zbuf, chunk,
                 sem_m0, sem_m1, sem_j0, sem_j1, sem_w0, sem_w1):
    cid = lax.axis_index("c")
    sid = lax.axis_index("s")
    wid = cid * _NS + sid
    base = wid * _TPW2
    lo = cid * _CRH
    zeros16 = jnp.zeros((16,), F32)
    jb = (jb0, jb1)
    db = (db0, db1)
    mr = (mr0, mr1)
    sem_m = (sem_m0, sem_m1)
    sem_j = (sem_j0, sem_j1)
    sem_w = (sem_w0, sem_w1)

    def zb_body(r2, carry):
        zbuf[r2, pl.ds(0, 16)] = zeros16
        return carry

    lax.fori_loop(0, _ZR, zb_body, 0)

    rows_zero = _CRHP // _NS
    rows_out = _CRH // _NS

    def quarter_body(q, carry0):
        for z in range(rows_zero // _ZR):
            pltpu.sync_copy(zbuf, chunk.at[pl.ds(sid * rows_zero + z * _ZR, _ZR)])
        plsc.subcore_barrier()

        for h, m_hbm in ((0, m0_hbm), (1, m1_hbm)):
            goff = h * _T2

            def issue_reads(b, par, drain):
                tb = base + b * _KA

                @pl.when(drain)
                def _():
                    pltpu.make_async_copy(mr[par], chunk.at[db[par]], sem_w[par]).wait()

                pltpu.async_copy(m_hbm.at[pl.ds(tb, _KA), pl.ds(q * 16, 16)], mr[par],
                                 sem_m[par])
                pltpu.async_copy(ji_hbm.at[pl.ds(goff + tb, _KA)], jb[par], sem_j[par])

            def do_batch(b, par):
                tb = base + b * _KA
                pltpu.make_async_copy(m_hbm.at[pl.ds(tb, _KA), pl.ds(q * 16, 16)],
                                      mr[par], sem_m[par]).wait()
                pltpu.make_async_copy(ji_hbm.at[pl.ds(goff + tb, _KA)], jb[par],
                                      sem_j[par]).wait()
                for g in range(_KA // 16):
                    ji = jb[par][pl.ds(g * 16, 16)]
                    rel = ji - lo
                    dst = jnp.where((rel >= 0) & (rel < _CRH), rel,
                                    _CRH + (ji & (_TRASH - 1)))
                    db[par][pl.ds(g * 16, 16)] = dst
                pltpu.async_copy(mr[par], chunk.at[db[par]], sem_w[par], add=True)

            issue_reads(0, 0, jnp.bool_(False))

            def pair_body(b2, carry):
                b = b2 * 2

                @pl.when(b + 1 < _NBA)
                def _():
                    issue_reads(b + 1, 1, b2 > 0)

                do_batch(b, 0)

                @pl.when(b + 2 < _NBA)
                def _():
                    issue_reads(b + 2, 0, jnp.bool_(True))

                @pl.when(b + 1 < _NBA)
                def _():
                    do_batch(b + 1, 1)

                return carry

            lax.fori_loop(0, (_NBA + 1) // 2, pair_body, 0)
            pltpu.make_async_copy(mr[0], chunk.at[db[0]], sem_w[0]).wait()
            pltpu.make_async_copy(mr[1], chunk.at[db[1]], sem_w[1]).wait()
        plsc.subcore_barrier()

        pltpu.sync_copy(chunk.at[pl.ds(sid * rows_out, rows_out)],
                        agg_hbm.at[pl.ds(lo + sid * rows_out, rows_out), pl.ds(q * 16, 16)])
        plsc.subcore_barrier()
        return carry0

    lax.fori_loop(0, I_DIM // 16, quarter_body, 0)


def _sc_agg(m0, m1, idx_ji):
    mesh = plsc.VectorSubcoreMesh(core_axis_name="c", subcore_axis_name="s")
    kern = functools.partial(
        pl.kernel,
        mesh=mesh,
        out_type=jax.ShapeDtypeStruct((_EP, 128), F32),
        compiler_params=pltpu.CompilerParams(use_tc_tiling_on_sc=False),
        scratch_types=[
            pltpu.VMEM((_KA,), I32),
            pltpu.VMEM((_KA,), I32),
            pltpu.VMEM((_KA,), I32),
            pltpu.VMEM((_KA,), I32),
            pltpu.VMEM((_KA, 16), F32),
            pltpu.VMEM((_KA, 16), F32),
            pltpu.VMEM((_ZR, 16), F32),
            pltpu.VMEM_SHARED((_CRHP, 16), F32),
            pltpu.SemaphoreType.DMA,
            pltpu.SemaphoreType.DMA,
            pltpu.SemaphoreType.DMA,
            pltpu.SemaphoreType.DMA,
            pltpu.SemaphoreType.DMA,
            pltpu.SemaphoreType.DMA,
        ],
    )(_sc_agg_body)
    return kern(m0, m1, idx_ji)


def _post_body(agg_ref, xji_ref, x_ref, wup, wbs1, bbs1, wbs2, bbs2,
               wlin, blin, wa1a, ba1a, wa1b, ba1b, wa2a, ba2a, wa2b, ba2b,
               out_ref):
    def mm(a, w, b=None):
        r = jnp.dot(a, w if isinstance(w, jax.Array) else w[...],
                    preferred_element_type=F32)
        if b is not None:
            r = r + b[...]
        return r

    h = xji_ref[...] + _silu(mm(agg_ref[...][:, :I_DIM], wup[...]))
    h = h + _silu(mm(_silu(mm(h, wbs1, bbs1)), wbs2, bbs2))
    h = _silu(mm(h, wlin, blin)) + x_ref[...]
    h = h + _silu(mm(_silu(mm(h, wa1a, ba1a)), wa1b, ba1b))
    h = h + _silu(mm(_silu(mm(h, wa2a, ba2a)), wa2b, ba2b))
    out_ref[...] = h


def _post(agg, xji, x, p):
    full = lambda shape: pl.BlockSpec(shape, lambda i: (0, 0))
    wspecs = []
    wvals = []
    for wn, bn in (('W_bs1', 'b_bs1'), ('W_bs2', 'b_bs2'), ('W_lin', 'b_lin'),
                   ('W_as1a', 'b_as1a'), ('W_as1b', 'b_as1b'),
                   ('W_as2a', 'b_as2a'), ('W_as2b', 'b_as2b')):
        wspecs += [full((H, H)), full((1, H))]
        wvals += [p[wn], p[bn].reshape(1, H)]
    return pl.pallas_call(
        _post_body,
        grid=(E // _BE,),
        in_specs=[
            pl.BlockSpec((_BE, 128), lambda i: (i, 0)),
            pl.BlockSpec((_BE, H), lambda i: (i, 0)),
            pl.BlockSpec((_BE, H), lambda i: (i, 0)),
            full((I_DIM, H)),
        ] + wspecs,
        out_specs=pl.BlockSpec((_BE, H), lambda i: (i, 0)),
        out_shape=jax.ShapeDtypeStruct((E, H), F32),
    )(agg, xji, x, p['W_up'], *wvals)


def kernel(x, rbf, sbf, idx_kj, idx_ji, params):
    xji, tbl = _pre(x, rbf, params)
    kj = idx_kj.astype(I32)
    s0 = _sfeat(sbf, params, 0)
    m0 = _sc_mul(tbl, s0, kj, 0)
    s1 = _sfeat(sbf, params, 1)
    m1 = _sc_mul(tbl, s1, kj, 1)
    agg = _sc_agg(m0, m1, idx_ji.astype(I32))
    return _post(agg, xji, x, params)


# speedup vs baseline: 1.2411x; 1.0902x over previous
import functools

import jax
import jax.numpy as jnp
from jax import lax
from jax.experimental import pallas as pl
from jax.experimental.pallas import tpu as pltpu
from jax.experimental.pallas import tpu_sc as plsc

E = 160000
T = 640000
H = 256
I_DIM = 64
NR = 6
SBF_DIM = 42

F32 = jnp.float32
I32 = jnp.int32


def _silu(v):
    return v * jax.nn.sigmoid(v)


_BE = 3200


def _pre_body(x_ref, rbf_ref, wji, bji, wkj, bkj, wr1, wr2, wdown, xji_out, tbl_out):
    xb = x_ref[...]
    x_ji = _silu(jnp.dot(xb, wji[...], preferred_element_type=F32) + bji[...])
    x_kj = _silu(jnp.dot(xb, wkj[...], preferred_element_type=F32) + bkj[...])
    r = jnp.dot(jnp.dot(rbf_ref[...], wr1[...], preferred_element_type=F32),
                wr2[...], preferred_element_type=F32)
    tbl = _silu(jnp.dot(x_kj * r, wdown[...], preferred_element_type=F32))
    xji_out[...] = x_ji
    tbl_out[...] = tbl


def _pre(x, rbf, p):
    full = lambda shape: pl.BlockSpec(shape, lambda i: (0, 0))
    return pl.pallas_call(
        _pre_body,
        grid=(E // _BE,),
        in_specs=[
            pl.BlockSpec((_BE, H), lambda i: (i, 0)),
            pl.BlockSpec((_BE, NR), lambda i: (i, 0)),
            full((H, H)), full((1, H)),
            full((H, H)), full((1, H)),
            full((NR, 8)), full((8, H)),
            full((H, I_DIM)),
        ],
        out_specs=[
            pl.BlockSpec((_BE, H), lambda i: (i, 0)),
            pl.BlockSpec((_BE, I_DIM), lambda i: (i, 0)),
        ],
        out_shape=[
            jax.ShapeDtypeStruct((E, H), F32),
            jax.ShapeDtypeStruct((E, I_DIM), F32),
        ],
    )(x, rbf, p['W_ji'], p['b_ji'].reshape(1, H), p['W_kj'], p['b_kj'].reshape(1, H),
      p['W_rbf1'], p['W_rbf2'], p['W_down'])


_BT = 4000


def _sfeat_body(sbf_ref, w1, w2, out_ref):
    out_ref[...] = jnp.dot(jnp.dot(sbf_ref[...], w1[...], preferred_element_type=F32),
                           w2[...], preferred_element_type=F32)


_T2 = T // 2


def _sfeat(sbf, p, part):
    nblk = _T2 // _BT
    return pl.pallas_call(
        _sfeat_body,
        grid=(nblk,),
        in_specs=[
            pl.BlockSpec((_BT, SBF_DIM), lambda i: (i + part * nblk, 0)),
            pl.BlockSpec((SBF_DIM, 8), lambda i: (0, 0)),
            pl.BlockSpec((8, 128), lambda i: (0, 0)),
        ],
        out_specs=pl.BlockSpec((_BT, 128), lambda i: (i, 0)),
        out_shape=jax.ShapeDtypeStruct((_T2, 128), F32),
    )(sbf, p['W_sbf1'],
      jnp.concatenate([p['W_sbf2'], jnp.zeros((8, 128 - I_DIM), F32)], axis=1))


_NC = 2
_NS = 16
_NW = _NC * _NS
_TPW = T // _NW
_TPW2 = _TPW // 2
_K = 200
_NB = _TPW2 // _K

_CRH = 81920
_TRASH = 512
_CRHP = _CRH + _TRASH
_EP = _NC * _CRH


def _sc_mul_body(part, tbl_hbm, s_hbm, kj_hbm, m_hbm,
                 kjbuf0, kjbuf1, trows0, trows1, srows0, srows1, prod0, prod1,
                 sem_i0, sem_i1, sem_g0, sem_g1, sem_s0, sem_s1, sem_w0, sem_w1):
    cid = lax.axis_index("c")
    sid = lax.axis_index("s")
    wid = cid * _NS + sid
    base = wid * _TPW2
    gbase = part * _T2 + base
    kjb = (kjbuf0, kjbuf1)
    trw = (trows0, trows1)
    srw = (srows0, srows1)
    prd = (prod0, prod1)
    sem_i = (sem_i0, sem_i1)
    sem_g = (sem_g0, sem_g1)
    sem_s = (sem_s0, sem_s1)
    sem_w = (sem_w0, sem_w1)

    def issue_reads(b, par):
        tb = base + b * _K
        pltpu.async_copy(kj_hbm.at[pl.ds(gbase + b * _K, _K)], kjb[par], sem_i[par]).wait()
        pltpu.async_copy(tbl_hbm.at[kjb[par]], trw[par], sem_g[par])
        pltpu.async_copy(s_hbm.at[pl.ds(tb, _K), pl.ds(0, I_DIM)], srw[par], sem_s[par])

    def do_batch(b, par, first, last):
        tb = base + b * _K
        pltpu.make_async_copy(tbl_hbm.at[kjb[par]], trw[par], sem_g[par]).wait()
        pltpu.make_async_copy(s_hbm.at[pl.ds(tb, _K), pl.ds(0, I_DIM)], srw[par],
                              sem_s[par]).wait()

        @pl.when(jnp.logical_not(first))
        def _():
            pltpu.make_async_copy(prd[par], m_hbm.at[pl.ds(0, _K)], sem_w[par]).wait()

        trows = trw[par]
        srows = srw[par]
        prod = prd[par]

        def mul_body(r4, carry2):
            rb = r4 * 4
            for dr in range(4):
                for c4 in range(I_DIM // 16):
                    sl = pl.ds(c4 * 16, 16)
                    prod[rb + dr, sl] = trows[rb + dr, sl] * srows[rb + dr, sl]
            return carry2

        lax.fori_loop(0, _K // 4, mul_body, 0)

        pltpu.async_copy(prod, m_hbm.at[pl.ds(tb, _K)], sem_w[par])

        @pl.when(last)
        def _():
            pltpu.make_async_copy(prd[par], m_hbm.at[pl.ds(0, _K)], sem_w[par]).wait()

    issue_reads(0, 0)

    def pair_body(b2, carry):
        b = b2 * 2

        @pl.when(b + 1 < _NB)
        def _():
            issue_reads(b + 1, 1)

        do_batch(b, 0, b == 0, b + 2 >= _NB)

        @pl.when(b + 2 < _NB)
        def _():
            issue_reads(b + 2, 0)

        @pl.when(b + 1 < _NB)
        def _():
            do_batch(b + 1, 1, b + 1 == 1, b + 3 >= _NB)

        return carry

    lax.fori_loop(0, (_NB + 1) // 2, pair_body, 0)


def _sc_mul(tbl, s, idx_kj, part):
    mesh = plsc.VectorSubcoreMesh(core_axis_name="c", subcore_axis_name="s")
    kern = functools.partial(
        pl.kernel,
        mesh=mesh,
        out_type=jax.ShapeDtypeStruct((_T2, I_DIM), F32),
        compiler_params=pltpu.CompilerParams(use_tc_tiling_on_sc=False),
        scratch_types=[
            pltpu.VMEM((_K,), I32),
            pltpu.VMEM((_K,), I32),
            pltpu.VMEM((_K, I_DIM), F32),
            pltpu.VMEM((_K, I_DIM), F32),
            pltpu.VMEM((_K, I_DIM), F32),
            pltpu.VMEM((_K, I_DIM), F32),
            pltpu.VMEM((_K, I_DIM), F32),
            pltpu.VMEM((_K, I_DIM), F32),
            pltpu.SemaphoreType.DMA,
            pltpu.SemaphoreType.DMA,
            pltpu.SemaphoreType.DMA,
            pltpu.SemaphoreType.DMA,
            pltpu.SemaphoreType.DMA,
            pltpu.SemaphoreType.DMA,
            pltpu.SemaphoreType.DMA,
            pltpu.SemaphoreType.DMA,
        ],
    )(functools.partial(_sc_mul_body, part))
    return kern(tbl, s, idx_kj)


_ZR = 644
_KA = 400
_NBA = _TPW2 // _KA


def _sc_agg_body(m0_hbm, m1_hbm, ji_hbm, agg_hbm,
                 jb0, jb1, db0, db1, mr0, mr1, zbuf, chunk,
                 sem_m0, sem_m1, sem_j0, sem_j1, sem_w0, sem_w1):
    cid = lax.axis_index("c")
    sid = lax.axis_index("s")
    wid = cid * _NS + sid
    base = wid * _TPW2
    lo = cid * _CRH
    zeros16 = jnp.zeros((16,), F32)
    jb = (jb0, jb1)
    db = (db0, db1)
    mr = (mr0, mr1)
    sem_m = (sem_m0, sem_m1)
    sem_j = (sem_j0, sem_j1)
    sem_w = (sem_w0, sem_w1)

    def zb_body(r2, carry):
        zbuf[r2, pl.ds(0, 16)] = zeros16
        return carry

    lax.fori_loop(0, _ZR, zb_body, 0)

    rows_zero = _CRHP // _NS
    rows_out = _CRH // _NS

    def quarter_body(q, carry0):
        for z in range(rows_zero // _ZR):
            pltpu.sync_copy(zbuf, chunk.at[pl.ds(sid * rows_zero + z * _ZR, _ZR)])
        plsc.subcore_barrier()

        for h, m_hbm in ((0, m0_hbm), (1, m1_hbm)):
            goff = h * _T2

            def issue_reads(b, par, drain):
                tb = base + b * _KA

                @pl.when(drain)
                def _():
                    pltpu.make_async_copy(mr[par], chunk.at[db[par]], sem_w[par]).wait()

                pltpu.async_copy(m_hbm.at[pl.ds(tb, _KA), pl.ds(q * 16, 16)], mr[par],
                                 sem_m[par])
                pltpu.async_copy(ji_hbm.at[pl.ds(goff + tb, _KA)], jb[par], sem_j[par])

            def do_batch(b, par):
                tb = base + b * _KA
                pltpu.make_async_copy(m_hbm.at[pl.ds(tb, _KA), pl.ds(q * 16, 16)],
                                      mr[par], sem_m[par]).wait()
                pltpu.make_async_copy(ji_hbm.at[pl.ds(goff + tb, _KA)], jb[par],
                                      sem_j[par]).wait()
                for g in range(_KA // 16):
                    ji = jb[par][pl.ds(g * 16, 16)]
                    rel = ji - lo
                    dst = jnp.where((rel >= 0) & (rel < _CRH), rel,
                                    _CRH + (ji & (_TRASH - 1)))
                    db[par][pl.ds(g * 16, 16)] = dst
                pltpu.async_copy(mr[par], chunk.at[db[par]], sem_w[par], add=True)

            issue_reads(0, 0, jnp.bool_(False))

            def pair_body(b2, carry):
                b = b2 * 2

                @pl.when(b + 1 < _NBA)
                def _():
                    issue_reads(b + 1, 1, b2 > 0)

                do_batch(b, 0)

                @pl.when(b + 2 < _NBA)
                def _():
                    issue_reads(b + 2, 0, jnp.bool_(True))

                @pl.when(b + 1 < _NBA)
                def _():
                    do_batch(b + 1, 1)

                return carry

            lax.fori_loop(0, (_NBA + 1) // 2, pair_body, 0)
            pltpu.make_async_copy(mr[0], chunk.at[db[0]], sem_w[0]).wait()
            pltpu.make_async_copy(mr[1], chunk.at[db[1]], sem_w[1]).wait()
        plsc.subcore_barrier()

        pltpu.sync_copy(chunk.at[pl.ds(sid * rows_out, rows_out)],
                        agg_hbm.at[pl.ds(lo + sid * rows_out, rows_out), pl.ds(q * 16, 16)])
        plsc.subcore_barrier()
        return carry0

    lax.fori_loop(0, I_DIM // 16, quarter_body, 0)


def _sc_agg(m0, m1, idx_ji):
    mesh = plsc.VectorSubcoreMesh(core_axis_name="c", subcore_axis_name="s")
    kern = functools.partial(
        pl.kernel,
        mesh=mesh,
        out_type=jax.ShapeDtypeStruct((_EP, 128), F32),
        compiler_params=pltpu.CompilerParams(use_tc_tiling_on_sc=False),
        scratch_types=[
            pltpu.VMEM((_KA,), I32),
            pltpu.VMEM((_KA,), I32),
            pltpu.VMEM((_KA,), I32),
            pltpu.VMEM((_KA,), I32),
            pltpu.VMEM((_KA, 16), F32),
            pltpu.VMEM((_KA, 16), F32),
            pltpu.VMEM((_ZR, 16), F32),
            pltpu.VMEM_SHARED((_CRHP, 16), F32),
            pltpu.SemaphoreType.DMA,
            pltpu.SemaphoreType.DMA,
            pltpu.SemaphoreType.DMA,
            pltpu.SemaphoreType.DMA,
            pltpu.SemaphoreType.DMA,
            pltpu.SemaphoreType.DMA,
        ],
    )(_sc_agg_body)
    return kern(m0, m1, idx_ji)


def _post_body(agg_ref, xji_ref, x_ref, wup, wbs1, bbs1, wbs2, bbs2,
               wlin, blin, wa1a, ba1a, wa1b, ba1b, wa2a, ba2a, wa2b, ba2b,
               out_ref):
    def mm(a, w, b=None):
        r = jnp.dot(a, w if isinstance(w, jax.Array) else w[...],
                    preferred_element_type=F32)
        if b is not None:
            r = r + b[...]
        return r

    h = xji_ref[...] + _silu(mm(agg_ref[...][:, :I_DIM], wup[...]))
    h = h + _silu(mm(_silu(mm(h, wbs1, bbs1)), wbs2, bbs2))
    h = _silu(mm(h, wlin, blin)) + x_ref[...]
    h = h + _silu(mm(_silu(mm(h, wa1a, ba1a)), wa1b, ba1b))
    h = h + _silu(mm(_silu(mm(h, wa2a, ba2a)), wa2b, ba2b))
    out_ref[...] = h


def _post(agg, xji, x, p):
    full = lambda shape: pl.BlockSpec(shape, lambda i: (0, 0))
    wspecs = []
    wvals = []
    for wn, bn in (('W_bs1', 'b_bs1'), ('W_bs2', 'b_bs2'), ('W_lin', 'b_lin'),
                   ('W_as1a', 'b_as1a'), ('W_as1b', 'b_as1b'),
                   ('W_as2a', 'b_as2a'), ('W_as2b', 'b_as2b')):
        wspecs += [full((H, H)), full((1, H))]
        wvals += [p[wn], p[bn].reshape(1, H)]
    return pl.pallas_call(
        _post_body,
        grid=(E // _BE,),
        in_specs=[
            pl.BlockSpec((_BE, 128), lambda i: (i, 0)),
            pl.BlockSpec((_BE, H), lambda i: (i, 0)),
            pl.BlockSpec((_BE, H), lambda i: (i, 0)),
            full((I_DIM, H)),
        ] + wspecs,
        out_specs=pl.BlockSpec((_BE, H), lambda i: (i, 0)),
        out_shape=jax.ShapeDtypeStruct((E, H), F32),
    )(agg, xji, x, p['W_up'], *wvals)


def kernel(x, rbf, sbf, idx_kj, idx_ji, params):
    xji, tbl = _pre(x, rbf, params)
    kj = idx_kj.astype(I32)
    s0 = _sfeat(sbf, params, 0)
    m0 = _sc_mul(tbl, s0, kj, 0)
    s1 = _sfeat(sbf, params, 1)
    m1 = _sc_mul(tbl, s1, kj, 1)
    agg = _sc_agg(m0, m1, idx_ji.astype(I32))
    return _post(agg, xji, x, params)


# speedup vs baseline: 1.2602x; 1.0154x over previous
import functools

import jax
import jax.numpy as jnp
from jax import lax
from jax.experimental import pallas as pl
from jax.experimental.pallas import tpu as pltpu
from jax.experimental.pallas import tpu_sc as plsc

E = 160000
T = 640000
H = 256
I_DIM = 64
NR = 6
SBF_DIM = 42

F32 = jnp.float32
I32 = jnp.int32


def _silu(v):
    return v * jax.nn.sigmoid(v)


_BE = 3200


def _pre_body(x_ref, rbf_ref, wji, bji, wkj, bkj, wr1, wr2, wdown, xji_out, tbl_out):
    xb = x_ref[...]
    x_ji = _silu(jnp.dot(xb, wji[...], preferred_element_type=F32) + bji[...])
    x_kj = _silu(jnp.dot(xb, wkj[...], preferred_element_type=F32) + bkj[...])
    r = jnp.dot(jnp.dot(rbf_ref[...], wr1[...], preferred_element_type=F32),
                wr2[...], preferred_element_type=F32)
    tbl = _silu(jnp.dot(x_kj * r, wdown[...], preferred_element_type=F32))
    xji_out[...] = x_ji
    tbl_out[...] = tbl


def _pre(x, rbf, p):
    full = lambda shape: pl.BlockSpec(shape, lambda i: (0, 0))
    return pl.pallas_call(
        _pre_body,
        grid=(E // _BE,),
        in_specs=[
            pl.BlockSpec((_BE, H), lambda i: (i, 0)),
            pl.BlockSpec((_BE, NR), lambda i: (i, 0)),
            full((H, H)), full((1, H)),
            full((H, H)), full((1, H)),
            full((NR, 8)), full((8, H)),
            full((H, I_DIM)),
        ],
        out_specs=[
            pl.BlockSpec((_BE, H), lambda i: (i, 0)),
            pl.BlockSpec((_BE, I_DIM), lambda i: (i, 0)),
        ],
        out_shape=[
            jax.ShapeDtypeStruct((E, H), F32),
            jax.ShapeDtypeStruct((E, I_DIM), F32),
        ],
    )(x, rbf, p['W_ji'], p['b_ji'].reshape(1, H), p['W_kj'], p['b_kj'].reshape(1, H),
      p['W_rbf1'], p['W_rbf2'], p['W_down'])


_BT = 8000


def _sfeat_body(sbf_ref, w1, w2, out_ref):
    out_ref[...] = jnp.dot(jnp.dot(sbf_ref[...], w1[...], preferred_element_type=F32),
                           w2[...], preferred_element_type=F32)


_T2 = T // 2


def _sfeat(sbf, p, part):
    nblk = _T2 // _BT
    return pl.pallas_call(
        _sfeat_body,
        grid=(nblk,),
        in_specs=[
            pl.BlockSpec((_BT, SBF_DIM), lambda i: (i + part * nblk, 0)),
            pl.BlockSpec((SBF_DIM, 8), lambda i: (0, 0)),
            pl.BlockSpec((8, 128), lambda i: (0, 0)),
        ],
        out_specs=pl.BlockSpec((_BT, 128), lambda i: (i, 0)),
        out_shape=jax.ShapeDtypeStruct((_T2, 128), F32),
    )(sbf, p['W_sbf1'],
      jnp.concatenate([p['W_sbf2'], jnp.zeros((8, 128 - I_DIM), F32)], axis=1))


_NC = 2
_NS = 16
_NW = _NC * _NS
_TPW = T // _NW
_TPW2 = _TPW // 2
_K = 200
_NB = _TPW2 // _K

_CRH = 81920
_TRASH = 512
_CRHP = _CRH + _TRASH
_EP = _NC * _CRH


def _sc_mul_body(part, tbl_hbm, s_hbm, kj_hbm, m_hbm,
                 kjbuf0, kjbuf1, trows0, trows1, srows0, srows1, prod0, prod1,
                 sem_i0, sem_i1, sem_g0, sem_g1, sem_s0, sem_s1, sem_w0, sem_w1):
    cid = lax.axis_index("c")
    sid = lax.axis_index("s")
    wid = cid * _NS + sid
    base = wid * _TPW2
    gbase = part * _T2 + base
    kjb = (kjbuf0, kjbuf1)
    trw = (trows0, trows1)
    srw = (srows0, srows1)
    prd = (prod0, prod1)
    sem_i = (sem_i0, sem_i1)
    sem_g = (sem_g0, sem_g1)
    sem_s = (sem_s0, sem_s1)
    sem_w = (sem_w0, sem_w1)

    def issue_reads(b, par):
        tb = base + b * _K
        pltpu.async_copy(kj_hbm.at[pl.ds(gbase + b * _K, _K)], kjb[par], sem_i[par]).wait()
        pltpu.async_copy(tbl_hbm.at[kjb[par]], trw[par], sem_g[par])
        pltpu.async_copy(s_hbm.at[pl.ds(tb, _K), pl.ds(0, I_DIM)], srw[par], sem_s[par])

    def do_batch(b, par, first, last):
        tb = base + b * _K
        pltpu.make_async_copy(tbl_hbm.at[kjb[par]], trw[par], sem_g[par]).wait()
        pltpu.make_async_copy(s_hbm.at[pl.ds(tb, _K), pl.ds(0, I_DIM)], srw[par],
                              sem_s[par]).wait()

        @pl.when(jnp.logical_not(first))
        def _():
            pltpu.make_async_copy(prd[par], m_hbm.at[pl.ds(0, _K)], sem_w[par]).wait()

        trows = trw[par]
        srows = srw[par]
        prod = prd[par]

        def mul_body(r4, carry2):
            rb = r4 * 4
            for dr in range(4):
                for c4 in range(I_DIM // 16):
                    sl = pl.ds(c4 * 16, 16)
                    prod[rb + dr, sl] = trows[rb + dr, sl] * srows[rb + dr, sl]
            return carry2

        lax.fori_loop(0, _K // 4, mul_body, 0)

        pltpu.async_copy(prod, m_hbm.at[pl.ds(tb, _K)], sem_w[par])

        @pl.when(last)
        def _():
            pltpu.make_async_copy(prd[par], m_hbm.at[pl.ds(0, _K)], sem_w[par]).wait()

    issue_reads(0, 0)

    def pair_body(b2, carry):
        b = b2 * 2

        @pl.when(b + 1 < _NB)
        def _():
            issue_reads(b + 1, 1)

        do_batch(b, 0, b == 0, b + 2 >= _NB)

        @pl.when(b + 2 < _NB)
        def _():
            issue_reads(b + 2, 0)

        @pl.when(b + 1 < _NB)
        def _():
            do_batch(b + 1, 1, b + 1 == 1, b + 3 >= _NB)

        return carry

    lax.fori_loop(0, (_NB + 1) // 2, pair_body, 0)


def _sc_mul(tbl, s, idx_kj, part):
    mesh = plsc.VectorSubcoreMesh(core_axis_name="c", subcore_axis_name="s")
    kern = functools.partial(
        pl.kernel,
        mesh=mesh,
        out_type=jax.ShapeDtypeStruct((_T2, I_DIM), F32),
        compiler_params=pltpu.CompilerParams(use_tc_tiling_on_sc=False),
        scratch_types=[
            pltpu.VMEM((_K,), I32),
            pltpu.VMEM((_K,), I32),
            pltpu.VMEM((_K, I_DIM), F32),
            pltpu.VMEM((_K, I_DIM), F32),
            pltpu.VMEM((_K, I_DIM), F32),
            pltpu.VMEM((_K, I_DIM), F32),
            pltpu.VMEM((_K, I_DIM), F32),
            pltpu.VMEM((_K, I_DIM), F32),
            pltpu.SemaphoreType.DMA,
            pltpu.SemaphoreType.DMA,
            pltpu.SemaphoreType.DMA,
            pltpu.SemaphoreType.DMA,
            pltpu.SemaphoreType.DMA,
            pltpu.SemaphoreType.DMA,
            pltpu.SemaphoreType.DMA,
            pltpu.SemaphoreType.DMA,
        ],
    )(functools.partial(_sc_mul_body, part))
    return kern(tbl, s, idx_kj)


_ZR = 644
_KA = 400
_NBA = _TPW2 // _KA


def _sc_agg_body(m0_hbm, m1_hbm, ji_hbm, agg_hbm,
                 jb0, jb1, db0, db1, mr0, mr1, zbuf, chunk,
                 sem_m0, sem_m1, sem_j0, sem_j1, sem_w0, sem_w1):
    cid = lax.axis_index("c")
    sid = lax.axis_index("s")
    wid = cid * _NS + sid
    base = wid * _TPW2
    lo = cid * _CRH
    zeros16 = jnp.zeros((16,), F32)
    jb = (jb0, jb1)
    db = (db0, db1)
    mr = (mr0, mr1)
    sem_m = (sem_m0, sem_m1)
    sem_j = (sem_j0, sem_j1)
    sem_w = (sem_w0, sem_w1)

    def zb_body(r2, carry):
        zbuf[r2, pl.ds(0, 16)] = zeros16
        return carry

    lax.fori_loop(0, _ZR, zb_body, 0)

    rows_zero = _CRHP // _NS
    rows_out = _CRH // _NS

    def quarter_body(q, carry0):
        for z in range(rows_zero // _ZR):
            pltpu.sync_copy(zbuf, chunk.at[pl.ds(sid * rows_zero + z * _ZR, _ZR)])
        plsc.subcore_barrier()

        for h, m_hbm in ((0, m0_hbm), (1, m1_hbm)):
            goff = h * _T2

            def issue_reads(b, par, drain):
                tb = base + b * _KA

                @pl.when(drain)
                def _():
                    pltpu.make_async_copy(mr[par], chunk.at[db[par]], sem_w[par]).wait()

                pltpu.async_copy(m_hbm.at[pl.ds(tb, _KA), pl.ds(q * 16, 16)], mr[par],
                                 sem_m[par])
                pltpu.async_copy(ji_hbm.at[pl.ds(goff + tb, _KA)], jb[par], sem_j[par])

            def do_batch(b, par):
                tb = base + b * _KA
                pltpu.make_async_copy(m_hbm.at[pl.ds(tb, _KA), pl.ds(q * 16, 16)],
                                      mr[par], sem_m[par]).wait()
                pltpu.make_async_copy(ji_hbm.at[pl.ds(goff + tb, _KA)], jb[par],
                                      sem_j[par]).wait()
                for g in range(_KA // 16):
                    ji = jb[par][pl.ds(g * 16, 16)]
                    rel = ji - lo
                    dst = jnp.where((rel >= 0) & (rel < _CRH), rel,
                                    _CRH + (ji & (_TRASH - 1)))
                    db[par][pl.ds(g * 16, 16)] = dst
                pltpu.async_copy(mr[par], chunk.at[db[par]], sem_w[par], add=True)

            issue_reads(0, 0, jnp.bool_(False))

            def pair_body(b2, carry):
                b = b2 * 2

                @pl.when(b + 1 < _NBA)
                def _():
                    issue_reads(b + 1, 1, b2 > 0)

                do_batch(b, 0)

                @pl.when(b + 2 < _NBA)
                def _():
                    issue_reads(b + 2, 0, jnp.bool_(True))

                @pl.when(b + 1 < _NBA)
                def _():
                    do_batch(b + 1, 1)

                return carry

            lax.fori_loop(0, (_NBA + 1) // 2, pair_body, 0)
            pltpu.make_async_copy(mr[0], chunk.at[db[0]], sem_w[0]).wait()
            pltpu.make_async_copy(mr[1], chunk.at[db[1]], sem_w[1]).wait()
        plsc.subcore_barrier()

        pltpu.sync_copy(chunk.at[pl.ds(sid * rows_out, rows_out)],
                        agg_hbm.at[pl.ds(lo + sid * rows_out, rows_out), pl.ds(q * 16, 16)])
        plsc.subcore_barrier()
        return carry0

    lax.fori_loop(0, I_DIM // 16, quarter_body, 0)


def _sc_agg(m0, m1, idx_ji):
    mesh = plsc.VectorSubcoreMesh(core_axis_name="c", subcore_axis_name="s")
    kern = functools.partial(
        pl.kernel,
        mesh=mesh,
        out_type=jax.ShapeDtypeStruct((_EP, 128), F32),
        compiler_params=pltpu.CompilerParams(use_tc_tiling_on_sc=False),
        scratch_types=[
            pltpu.VMEM((_KA,), I32),
            pltpu.VMEM((_KA,), I32),
            pltpu.VMEM((_KA,), I32),
            pltpu.VMEM((_KA,), I32),
            pltpu.VMEM((_KA, 16), F32),
            pltpu.VMEM((_KA, 16), F32),
            pltpu.VMEM((_ZR, 16), F32),
            pltpu.VMEM_SHARED((_CRHP, 16), F32),
            pltpu.SemaphoreType.DMA,
            pltpu.SemaphoreType.DMA,
            pltpu.SemaphoreType.DMA,
            pltpu.SemaphoreType.DMA,
            pltpu.SemaphoreType.DMA,
            pltpu.SemaphoreType.DMA,
        ],
    )(_sc_agg_body)
    return kern(m0, m1, idx_ji)


def _post_body(agg_ref, xji_ref, x_ref, wup, wbs1, bbs1, wbs2, bbs2,
               wlin, blin, wa1a, ba1a, wa1b, ba1b, wa2a, ba2a, wa2b, ba2b,
               out_ref):
    def mm(a, w, b=None):
        r = jnp.dot(a, w if isinstance(w, jax.Array) else w[...],
                    preferred_element_type=F32)
        if b is not None:
            r = r + b[...]
        return r

    h = xji_ref[...] + _silu(mm(agg_ref[...][:, :I_DIM], wup[...]))
    h = h + _silu(mm(_silu(mm(h, wbs1, bbs1)), wbs2, bbs2))
    h = _silu(mm(h, wlin, blin)) + x_ref[...]
    h = h + _silu(mm(_silu(mm(h, wa1a, ba1a)), wa1b, ba1b))
    h = h + _silu(mm(_silu(mm(h, wa2a, ba2a)), wa2b, ba2b))
    out_ref[...] = h


def _post(agg, xji, x, p):
    full = lambda shape: pl.BlockSpec(shape, lambda i: (0, 0))
    wspecs = []
    wvals = []
    for wn, bn in (('W_bs1', 'b_bs1'), ('W_bs2', 'b_bs2'), ('W_lin', 'b_lin'),
                   ('W_as1a', 'b_as1a'), ('W_as1b', 'b_as1b'),
                   ('W_as2a', 'b_as2a'), ('W_as2b', 'b_as2b')):
        wspecs += [full((H, H)), full((1, H))]
        wvals += [p[wn], p[bn].reshape(1, H)]
    return pl.pallas_call(
        _post_body,
        grid=(E // _BE,),
        in_specs=[
            pl.BlockSpec((_BE, 128), lambda i: (i, 0)),
            pl.BlockSpec((_BE, H), lambda i: (i, 0)),
            pl.BlockSpec((_BE, H), lambda i: (i, 0)),
            full((I_DIM, H)),
        ] + wspecs,
        out_specs=pl.BlockSpec((_BE, H), lambda i: (i, 0)),
        out_shape=jax.ShapeDtypeStruct((E, H), F32),
    )(agg, xji, x, p['W_up'], *wvals)


def kernel(x, rbf, sbf, idx_kj, idx_ji, params):
    xji, tbl = _pre(x, rbf, params)
    kj = idx_kj.astype(I32)
    s0 = _sfeat(sbf, params, 0)
    m0 = _sc_mul(tbl, s0, kj, 0)
    s1 = _sfeat(sbf, params, 1)
    m1 = _sc_mul(tbl, s1, kj, 1)
    agg = _sc_agg(m0, m1, idx_ji.astype(I32))
    return _post(agg, xji, x, params)
